# baseline (speedup 1.0000x reference)
import functools

import jax
import jax.numpy as jnp
from jax import lax
from jax.experimental import pallas as pl
from jax.experimental.pallas import tpu as pltpu
from jax.experimental.pallas import tpu_sc as plsc

N = 100000
H = 128
V = 100

NC = 2
NS = 16
NW = NC * NS
CHUNK = 128
CPW = 25
NBUF = 7
LAG = 3
LAST = N - CHUNK


def _embed_sc(z, table):
    mesh = plsc.VectorSubcoreMesh(core_axis_name="c", subcore_axis_name="s")

    @functools.partial(
        pl.kernel,
        out_type=jax.ShapeDtypeStruct((N, H), jnp.float32),
        mesh=mesh,
        scratch_types=[
            pltpu.VMEM((NBUF, CHUNK), jnp.int32),
            pltpu.VMEM((NBUF, CHUNK, H), jnp.float32),
            pltpu.VMEM_SHARED((V, H), jnp.float32),
            pltpu.SemaphoreType.DMA,
            pltpu.SemaphoreType.DMA((NBUF,)),
            pltpu.SemaphoreType.DMA((NBUF,)),
            pltpu.SemaphoreType.DMA((NBUF,)),
        ],
    )
    def k(z_hbm, table_hbm, out_hbm, idx_v, rows_v, table_sh, sem_t, sem_i,
          sem_g, sem_w):
        sid = lax.axis_index("s")
        wid = sid * NC + lax.axis_index("c")

        @pl.when(sid == 0)
        def _():
            pltpu.async_copy(table_hbm, table_sh, sem_t)

        def off_of(j):
            return lax.min((wid + j * NW) * CHUNK, LAST)

        def fire_idx(j, b):
            pltpu.async_copy(z_hbm.at[pl.ds(off_of(j), CHUNK)],
                             idx_v.at[b], sem_i.at[b])

        def wait_i(b):
            pltpu.make_async_copy(z_hbm.at[pl.ds(0, CHUNK)],
                                  idx_v.at[b], sem_i.at[b]).wait()

        def prol(j, c):
            fire_idx(j, j)
            return c
        lax.fori_loop(0, min(NBUF, CPW), prol, 0)

        @pl.when(sid == 0)
        def _():
            pltpu.make_async_copy(table_hbm, table_sh, sem_t).wait()
        plsc.subcore_barrier()

        def gather(j, b):
            pltpu.async_copy(table_sh.at[idx_v.at[b]], rows_v.at[b],
                             sem_g.at[b])

        def write(j, b):
            pltpu.async_copy(rows_v.at[b],
                             out_hbm.at[pl.ds(off_of(j), CHUNK)],
                             sem_w.at[b])

        def wait_g(b):
            pltpu.make_async_copy(table_sh.at[idx_v.at[0]], rows_v.at[b],
                                  sem_g.at[b]).wait()

        def wait_w(b):
            pltpu.make_async_copy(rows_v.at[b],
                                  out_hbm.at[pl.ds(0, CHUNK)],
                                  sem_w.at[b]).wait()

        def body(j, c):
            b = lax.rem(j, NBUF)

            @pl.when(j >= NBUF)
            def _():
                wait_w(b)

            wait_i(b)
            gather(j, b)

            @pl.when(j >= LAG)
            def _():
                bp = lax.rem(j - LAG, NBUF)
                wait_g(bp)
                write(j - LAG, bp)

                @pl.when(j - LAG + NBUF < CPW)
                def _():
                    fire_idx(j - LAG + NBUF, bp)
            return c

        lax.fori_loop(0, CPW, body, 0)

        def tail(j, c):
            b = lax.rem(j, NBUF)
            wait_g(b)
            write(j, b)
            return c
        lax.fori_loop(CPW - LAG, CPW, tail, 0)

        def drain_w(j, c):
            wait_w(lax.rem(j, NBUF))
            return c
        lax.fori_loop(CPW - NBUF, CPW, drain_w, 0)

    return k(z, table)


def kernel(atomic_numbers, pos, batch, edge_index, cell, cell_offsets,
           neighbors, W):
    z = atomic_numbers.astype(jnp.int32)
    h = _embed_sc(z, W.astype(jnp.float32))
    return (h, atomic_numbers, pos, batch, edge_index, cell, cell_offsets,
            neighbors)


# speedup vs baseline: 1.0185x; 1.0185x over previous
import functools

import jax
import jax.numpy as jnp
from jax import lax
from jax.experimental import pallas as pl
from jax.experimental.pallas import tpu as pltpu
from jax.experimental.pallas import tpu_sc as plsc

N = 100000
H = 128
V = 100

NC = 2
NS = 16
NW = NC * NS
CHUNK = 128
CPW = 25
NBUF = 7
LAG = 2
LAST = N - CHUNK


def _embed_sc(z, table):
    mesh = plsc.VectorSubcoreMesh(core_axis_name="c", subcore_axis_name="s")

    @functools.partial(
        pl.kernel,
        out_type=jax.ShapeDtypeStruct((N, H), jnp.float32),
        mesh=mesh,
        scratch_types=[
            pltpu.VMEM((CPW, CHUNK), jnp.int32),
            pltpu.VMEM((NBUF, CHUNK, H), jnp.float32),
            pltpu.VMEM_SHARED((V, H), jnp.float32),
            pltpu.SemaphoreType.DMA,
            pltpu.SemaphoreType.DMA,
            pltpu.SemaphoreType.DMA((NBUF,)),
            pltpu.SemaphoreType.DMA((NBUF,)),
        ],
    )
    def k(z_hbm, table_hbm, out_hbm, idx_v, rows_v, table_sh, sem_t, sem_i,
          sem_g, sem_w):
        sid = lax.axis_index("s")
        wid = sid * NC + lax.axis_index("c")

        @pl.when(sid == 0)
        def _():
            pltpu.async_copy(table_hbm, table_sh, sem_t)

        def off_of(j):
            return lax.min((wid + j * NW) * CHUNK, LAST)

        def fire_idx(j, c):
            pltpu.async_copy(z_hbm.at[pl.ds(off_of(j), CHUNK)],
                             idx_v.at[j], sem_i)
            return c
        lax.fori_loop(0, CPW, fire_idx, 0)

        def drain_idx(j, c):
            pltpu.make_async_copy(z_hbm.at[pl.ds(0, CHUNK)],
                                  idx_v.at[0], sem_i).wait()
            return c
        lax.fori_loop(0, CPW, drain_idx, 0)

        @pl.when(sid == 0)
        def _():
            pltpu.make_async_copy(table_hbm, table_sh, sem_t).wait()
        plsc.subcore_barrier()

        def gather(j, b):
            pltpu.async_copy(table_sh.at[idx_v.at[j]], rows_v.at[b],
                             sem_g.at[b])

        def write(j, b):
            pltpu.async_copy(rows_v.at[b],
                             out_hbm.at[pl.ds(off_of(j), CHUNK)],
                             sem_w.at[b])

        def wait_g(b):
            pltpu.make_async_copy(table_sh.at[idx_v.at[0]], rows_v.at[b],
                                  sem_g.at[b]).wait()

        def wait_w(b):
            pltpu.make_async_copy(rows_v.at[b],
                                  out_hbm.at[pl.ds(0, CHUNK)],
                                  sem_w.at[b]).wait()

        def body(j, c):
            b = lax.rem(j, NBUF)

            @pl.when(j >= NBUF)
            def _():
                wait_w(b)

            gather(j, b)

            @pl.when(j >= LAG)
            def _():
                bp = lax.rem(j - LAG, NBUF)
                wait_g(bp)
                write(j - LAG, bp)
            return c

        lax.fori_loop(0, CPW, body, 0)

        def tail(j, c):
            b = lax.rem(j, NBUF)
            wait_g(b)
            write(j, b)
            return c
        lax.fori_loop(CPW - LAG, CPW, tail, 0)

        def drain_w(j, c):
            wait_w(lax.rem(j, NBUF))
            return c
        lax.fori_loop(CPW - NBUF, CPW, drain_w, 0)

    return k(z, table)


def kernel(atomic_numbers, pos, batch, edge_index, cell, cell_offsets,
           neighbors, W):
    z = atomic_numbers.astype(jnp.int32)
    h = _embed_sc(z, W.astype(jnp.float32))
    return (h, atomic_numbers, pos, batch, edge_index, cell, cell_offsets,
            neighbors)
